# baseline (speedup 1.0000x reference)
import functools

import jax
import jax.numpy as jnp
from jax import lax
from jax.experimental import pallas as pl
from jax.experimental.pallas import tpu as pltpu
from jax.experimental.pallas import tpu_sc as plsc

NSITES = 16384
D = 128
NC = 2
NS = 16
NW = NC * NS
CPW = NSITES // NW
S = 128
NSUB = CPW // S
L = 16

_mesh = plsc.VectorSubcoreMesh(core_axis_name="c", subcore_axis_name="s")


@functools.partial(
    pl.kernel,
    out_type=jax.ShapeDtypeStruct((D, NSITES), jnp.float32),
    mesh=_mesh,
    compiler_params=pltpu.CompilerParams(needs_layout_passes=False),
    scratch_types=[
        pltpu.VMEM((S,), jnp.float32),
        pltpu.VMEM((S,), jnp.float32),
        pltpu.VMEM((S * D,), jnp.float32),
        pltpu.VMEM((D, S), jnp.float32),
        pltpu.VMEM((4 * D,), jnp.float32),
    ],
)
def _sc_body(x_hbm, e_hbm, p_hbm, out_hbm, x0_v, x1_v, p_v, pt_v, e_v):
    cid = lax.axis_index("c")
    sid = lax.axis_index("s")
    wid = sid * NC + cid
    base = wid * CPW
    pltpu.sync_copy(e_hbm, e_v)
    iota = lax.iota(jnp.int32, L)
    zf = jnp.zeros((L,), jnp.float32)
    zi = jnp.zeros((L,), jnp.int32)
    onei = jnp.ones((L,), jnp.int32)
    twoi = jnp.full((L,), 2, jnp.int32)
    cd = jnp.full((L,), D, jnp.int32)
    for k in range(NSUB):
        sbase = base + k * S
        pltpu.sync_copy(x_hbm.at[0, pl.ds(sbase, S)], x0_v)
        pltpu.sync_copy(x_hbm.at[1, pl.ds(sbase, S)], x1_v)
        pltpu.sync_copy(p_hbm.at[pl.ds(sbase * D, S * D)], p_v)
        for j0 in range(0, S, L):
            v0 = x0_v[pl.ds(j0, L)]
            v1 = x1_v[pl.ds(j0, L)]
            idx16 = jnp.where(v0 < zf, twoi, zi) + jnp.where(v1 < zf, onei, zi)
            eoff = idx16 * cd
            rows = iota + jnp.full((L,), j0, jnp.int32)
            poff = rows * cd

            def dbody(d, carry, rows=rows, eoff=eoff, poff=poff):
                dd = lax.broadcast_in_dim(d, (L,), ())
                g = plsc.load_gather(p_v, [poff + dd])
                e = plsc.load_gather(e_v, [eoff + dd])
                plsc.store_scatter(pt_v, [dd, rows], g + e)
                return carry

            lax.fori_loop(0, D, dbody, 0, unroll=4)
        pltpu.sync_copy(pt_v, out_hbm.at[:, pl.ds(sbase, S)])


def kernel(x, E, P):
    return _sc_body(x, E.reshape(-1), P.reshape(-1))


# speedup vs baseline: 1.3612x; 1.3612x over previous
import functools

import jax
import jax.numpy as jnp
from jax import lax
from jax.experimental import pallas as pl
from jax.experimental.pallas import tpu as pltpu
from jax.experimental.pallas import tpu_sc as plsc

NSITES = 16384
D = 128
NC = 2
NS = 16
NW = NC * NS
CPW = NSITES // NW
S = 128
NSUB = CPW // S
L = 16
PP = 129

_mesh = plsc.VectorSubcoreMesh(core_axis_name="c", subcore_axis_name="s")


@functools.partial(
    pl.kernel,
    out_type=jax.ShapeDtypeStruct((D, NSITES), jnp.float32),
    mesh=_mesh,
    compiler_params=pltpu.CompilerParams(needs_layout_passes=False),
    scratch_types=[
        pltpu.VMEM((S,), jnp.float32),
        pltpu.VMEM((S,), jnp.float32),
        pltpu.VMEM((S, PP), jnp.float32),
        pltpu.VMEM((D, S), jnp.float32),
        pltpu.VMEM((D * L,), jnp.float32),
    ],
)
def _sc_body(x_hbm, et_hbm, p_hbm, out_hbm, x0_v, x1_v, p_v, pt_v, et_v):
    cid = lax.axis_index("c")
    sid = lax.axis_index("s")
    wid = sid * NC + cid
    base = wid * CPW
    pltpu.sync_copy(et_hbm, et_v)
    iota = lax.iota(jnp.int32, L)
    zf = jnp.zeros((L,), jnp.float32)
    zi = jnp.zeros((L,), jnp.int32)
    onei = jnp.ones((L,), jnp.int32)
    twoi = jnp.full((L,), 2, jnp.int32)
    for k in range(NSUB):
        sbase = base + k * S
        pltpu.sync_copy(x_hbm.at[0, pl.ds(sbase, S)], x0_v)
        pltpu.sync_copy(x_hbm.at[1, pl.ds(sbase, S)], x1_v)
        pltpu.sync_copy(
            p_hbm.at[pl.ds(sbase, S), :],
            p_v.at[:, pl.ds(0, D)],
        )
        for j0 in range(0, S, L):
            v0 = x0_v[pl.ds(j0, L)]
            v1 = x1_v[pl.ds(j0, L)]
            idx16 = jnp.where(v0 < zf, twoi, zi) + jnp.where(v1 < zf, onei, zi)
            rows = iota + jnp.full((L,), j0, jnp.int32)

            def dbody(d, carry, rows=rows, idx16=idx16):
                dd = lax.broadcast_in_dim(d, (L,), ())
                g = plsc.load_gather(p_v, [rows, dd])
                e = plsc.load_gather(et_v, [dd * jnp.full((L,), L, jnp.int32) + idx16])
                plsc.store_scatter(pt_v, [dd, rows], g + e)
                return carry

            lax.fori_loop(0, D, dbody, 0, unroll=8)
        pltpu.sync_copy(pt_v, out_hbm.at[:, pl.ds(sbase, S)])


def kernel(x, E, P):
    etp = jnp.pad(E.T, ((0, 0), (0, L - 4))).reshape(-1)
    return _sc_body(x, etp, P)


# speedup vs baseline: 1.4592x; 1.0720x over previous
import functools

import jax
import jax.numpy as jnp
from jax import lax
from jax.experimental import pallas as pl
from jax.experimental.pallas import tpu as pltpu
from jax.experimental.pallas import tpu_sc as plsc

NSITES = 16384
D = 128
NC = 2
NS = 16
NW = NC * NS
CPW = NSITES // NW
S = 128
NSUB = CPW // S
L = 16
PP = D + 1
EP = 4 * D + 1

_mesh = plsc.VectorSubcoreMesh(core_axis_name="c", subcore_axis_name="s")


@functools.partial(
    pl.kernel,
    out_type=jax.ShapeDtypeStruct((D, NSITES), jnp.float32),
    mesh=_mesh,
    compiler_params=pltpu.CompilerParams(needs_layout_passes=False),
    scratch_types=[
        pltpu.VMEM((2, S), jnp.float32),
        pltpu.VMEM((2, S), jnp.float32),
        pltpu.VMEM((S, PP), jnp.float32),
        pltpu.VMEM((S, PP), jnp.float32),
        pltpu.VMEM((D, S), jnp.float32),
        pltpu.VMEM((D, S), jnp.float32),
        pltpu.VMEM((L * EP,), jnp.float32),
        pltpu.SemaphoreType.DMA,
        pltpu.SemaphoreType.DMA,
        pltpu.SemaphoreType.DMA,
        pltpu.SemaphoreType.DMA,
    ],
)
def _sc_body(x_hbm, et_hbm, p_hbm, out_hbm,
             x_v0, x_v1, p_v0, p_v1, pt_v0, pt_v1, et_v,
             in_s0, in_s1, out_s0, out_s1):
    cid = lax.axis_index("c")
    sid = lax.axis_index("s")
    wid = sid * NC + cid
    base = wid * CPW
    pltpu.sync_copy(et_hbm, et_v)

    xv = (x_v0, x_v1)
    pv = (p_v0, p_v1)
    ptv = (pt_v0, pt_v1)
    ins = (in_s0, in_s1)
    outs = (out_s0, out_s1)

    iota = lax.iota(jnp.int32, L)
    lane_base = iota * jnp.full((L,), EP, jnp.int32)
    zf = jnp.zeros((L,), jnp.float32)
    zi = jnp.zeros((L,), jnp.int32)
    onei = jnp.ones((L,), jnp.int32)
    twoi = jnp.full((L,), 2, jnp.int32)
    four = jnp.full((L,), 4, jnp.int32)

    def start_in(k):
        b = k % 2
        sb = base + k * S
        d1 = pltpu.async_copy(x_hbm.at[:, pl.ds(sb, S)], xv[b], ins[b])
        d2 = pltpu.async_copy(
            p_hbm.at[pl.ds(sb, S), :], pv[b].at[:, pl.ds(0, D)], ins[b])
        return (d1, d2)

    pending_in = {0: start_in(0), 1: None}
    pending_out = {0: None, 1: None}
    for k in range(NSUB):
        b = k % 2
        if k + 1 < NSUB:
            pending_in[1 - b] = start_in(k + 1)
        for dsc in pending_in[b]:
            dsc.wait()
        if pending_out[b] is not None:
            pending_out[b].wait()
        for j0 in range(0, S, L):
            v0 = xv[b][0, pl.ds(j0, L)]
            v1 = xv[b][1, pl.ds(j0, L)]
            idx16 = jnp.where(v0 < zf, twoi, zi) + jnp.where(v1 < zf, onei, zi)
            e_base = lane_base + idx16
            rows = iota + jnp.full((L,), j0, jnp.int32)

            def dbody(d, carry, rows=rows, e_base=e_base, b=b):
                dd = lax.broadcast_in_dim(d, (L,), ())
                g = plsc.load_gather(pv[b], [rows, dd])
                e = plsc.load_gather(et_v, [e_base + dd * four])
                plsc.store_scatter(ptv[b], [dd, rows], g + e)
                return carry

            lax.fori_loop(0, D, dbody, 0, unroll=8)
        sb = base + k * S
        pending_out[b] = pltpu.async_copy(
            ptv[b], out_hbm.at[:, pl.ds(sb, S)], outs[b])
    for b in range(2):
        if pending_out[b] is not None:
            pending_out[b].wait()


def kernel(x, E, P):
    et = jnp.pad(E.T.reshape(-1), (0, EP - 4 * D))
    etr = jnp.tile(et, (L,))
    return _sc_body(x, etr, P)


# speedup vs baseline: 2.4186x; 1.6575x over previous
import functools

import jax
import jax.numpy as jnp
from jax import lax
from jax.experimental import pallas as pl
from jax.experimental.pallas import tpu as pltpu
from jax.experimental.pallas import tpu_sc as plsc

NSITES = 16384
D = 128
FSC = 16
FTC = D - FSC
NC = 2
NS = 16
NW = NC * NS
CPW = NSITES // NW
S = 128
NSUB = CPW // S
L = 16
PP = D + 1
EP = 4 * FSC + 1
BN = 2048

_mesh = plsc.VectorSubcoreMesh(core_axis_name="c", subcore_axis_name="s")


@functools.partial(
    pl.kernel,
    out_type=jax.ShapeDtypeStruct((FSC, NSITES), jnp.float32),
    mesh=_mesh,
    compiler_params=pltpu.CompilerParams(needs_layout_passes=False),
    scratch_types=[
        pltpu.VMEM((2, S), jnp.float32),
        pltpu.VMEM((2, S), jnp.float32),
        pltpu.VMEM((S, PP), jnp.float32),
        pltpu.VMEM((S, PP), jnp.float32),
        pltpu.VMEM((FSC, S), jnp.float32),
        pltpu.VMEM((FSC, S), jnp.float32),
        pltpu.VMEM((L * EP,), jnp.float32),
        pltpu.SemaphoreType.DMA,
        pltpu.SemaphoreType.DMA,
        pltpu.SemaphoreType.DMA,
        pltpu.SemaphoreType.DMA,
    ],
)
def _sc_body(x_hbm, et_hbm, p_hbm, out_hbm,
             x_v0, x_v1, p_v0, p_v1, pt_v0, pt_v1, et_v,
             in_s0, in_s1, out_s0, out_s1):
    cid = lax.axis_index("c")
    sid = lax.axis_index("s")
    wid = sid * NC + cid
    base = wid * CPW
    pltpu.sync_copy(et_hbm, et_v)

    xv = (x_v0, x_v1)
    pv = (p_v0, p_v1)
    ptv = (pt_v0, pt_v1)
    ins = (in_s0, in_s1)
    outs = (out_s0, out_s1)

    iota = lax.iota(jnp.int32, L)
    lane_base = iota * jnp.full((L,), EP, jnp.int32)
    zf = jnp.zeros((L,), jnp.float32)
    zi = jnp.zeros((L,), jnp.int32)
    onei = jnp.ones((L,), jnp.int32)
    twoi = jnp.full((L,), 2, jnp.int32)

    def start_in(k):
        b = k % 2
        sb = base + k * S
        d1 = pltpu.async_copy(x_hbm.at[:, pl.ds(sb, S)], xv[b], ins[b])
        d2 = pltpu.async_copy(
            p_hbm.at[pl.ds(sb, S), :], pv[b].at[:, pl.ds(0, D)], ins[b])
        return (d1, d2)

    pending_in = {0: start_in(0), 1: None}
    pending_out = {0: None, 1: None}
    for k in range(NSUB):
        b = k % 2
        if k + 1 < NSUB:
            pending_in[1 - b] = start_in(k + 1)
        for dsc in pending_in[b]:
            dsc.wait()
        if pending_out[b] is not None:
            pending_out[b].wait()
        for j0 in range(0, S, L):
            v0 = xv[b][0, pl.ds(j0, L)]
            v1 = xv[b][1, pl.ds(j0, L)]
            idx16 = jnp.where(v0 < zf, twoi, zi) + jnp.where(v1 < zf, onei, zi)
            e_base = lane_base + idx16
            rows = iota + jnp.full((L,), j0, jnp.int32)
            for d in range(FSC):
                dd = jnp.full((L,), d, jnp.int32)
                g = plsc.load_gather(pv[b], [rows, dd])
                e = plsc.load_gather(et_v, [e_base + dd * 4])
                plsc.store_scatter(ptv[b], [dd, rows], g + e)
        sb = base + k * S
        pending_out[b] = pltpu.async_copy(
            ptv[b], out_hbm.at[:, pl.ds(sb, S)], outs[b])
    for b in range(2):
        if pending_out[b] is not None:
            pending_out[b].wait()


def _tc_body(x_ref, et_ref, p_ref, o_ref):
    pt = p_ref[:].T[FSC:, :]
    b0 = x_ref[0:1, :] < 0.0
    b1 = x_ref[1:2, :] < 0.0
    et = et_ref[:]
    e0 = et[:, 0:1]
    e1 = et[:, 1:2]
    e2 = et[:, 2:3]
    e3 = et[:, 3:4]
    sel = jnp.where(b0, jnp.where(b1, e3, e2), jnp.where(b1, e1, e0))
    o_ref[:] = pt + sel


def kernel(x, E, P):
    et = E.T
    esub = jnp.pad(et[:FSC, :].reshape(-1), (0, EP - 4 * FSC))
    etr = jnp.tile(esub, (L,))
    sc_out = _sc_body(x, etr, P)

    tc_out = pl.pallas_call(
        _tc_body,
        grid=(NSITES // BN,),
        in_specs=[
            pl.BlockSpec((2, BN), lambda i: (0, i)),
            pl.BlockSpec((FTC, 4), lambda i: (0, 0)),
            pl.BlockSpec((BN, D), lambda i: (i, 0)),
        ],
        out_specs=pl.BlockSpec((FTC, BN), lambda i: (0, i)),
        out_shape=jax.ShapeDtypeStruct((FTC, NSITES), jnp.float32),
    )(x, et[FSC:], P)

    return jnp.concatenate([sc_out, tc_out], axis=0)
